# baseline (speedup 1.0000x reference)
import functools

import jax
import jax.numpy as jnp
from jax import lax
from jax.experimental import pallas as pl
from jax.experimental.pallas import tpu as pltpu
from jax.experimental.pallas import tpu_sc as plsc

N = 10000
E = 320000
NC = 2
NS = 16
K = 128
CHUNK_ROWS = 2560
EPAD = CHUNK_ROWS * K - E
NTRASH = 128
ACC_ROWS = N + NTRASH
RPT = 624
DEGW = 128

BR = 1000
GRID = N // BR

_mesh = plsc.VectorSubcoreMesh(
    core_axis_name="c", subcore_axis_name="s", num_cores=NC, num_subcores=NS)


def _tilewise_copy(src_at, dst_at, s):
  off = pl.multiple_of(s * RPT, 8)
  pltpu.sync_copy(src_at(off, RPT), dst_at(off, RPT))

  @pl.when(s == NS - 1)
  def _():
    pltpu.sync_copy(src_at(NS * RPT, N - NS * RPT), dst_at(NS * RPT, N - NS * RPT))


RB = 40


def _edge_pipeline(start_row, rows_per_tile, y_hbm, src2_hbm, dst2_hbm,
                   srcv, dstv, rows2, acc, gsems, ssems):
  rows0, rows1 = rows2
  gs0, gs1 = gsems
  ss0, ss1 = ssems

  def g_issue(j, buf, sem):
    pltpu.async_copy(y_hbm.at[srcv.at[j]], buf, sem)

  def g_wait(buf, sem):
    pltpu.make_async_copy(y_hbm.at[srcv.at[0]], buf, sem).wait()

  def s_issue(j, buf, sem):
    pltpu.async_copy(buf, acc.at[dstv.at[j]], sem, add=True)

  def s_wait(buf, sem):
    pltpu.make_async_copy(buf, acc.at[dstv.at[0]], sem).wait()

  def blk(b, carry):
    r0 = pl.multiple_of(start_row + b * RB, 8)
    pltpu.sync_copy(src2_hbm.at[pl.ds(r0, RB)], srcv)
    pltpu.sync_copy(dst2_hbm.at[pl.ds(r0, RB)], dstv)
    g_issue(0, rows0, gs0)
    g_issue(1, rows1, gs1)
    g_wait(rows0, gs0)
    s_issue(0, rows0, ss0)
    g_wait(rows1, gs1)
    s_issue(1, rows1, ss1)

    def pair(i, c):
      j0 = i * 2
      s_wait(rows0, ss0)
      g_issue(j0, rows0, gs0)
      s_wait(rows1, ss1)
      g_issue(j0 + 1, rows1, gs1)
      g_wait(rows0, gs0)
      s_issue(j0, rows0, ss0)
      g_wait(rows1, gs1)
      s_issue(j0 + 1, rows1, ss1)
      return c

    lax.fori_loop(1, RB // 2, pair, 0)
    s_wait(rows0, ss0)
    s_wait(rows1, ss1)
    return carry

  lax.fori_loop(0, rows_per_tile // RB, blk, 0)


def _prop_split():
  rpt_edges = CHUNK_ROWS // NS

  @functools.partial(
      pl.kernel,
      out_type=jax.ShapeDtypeStruct((NC * N, 128), jnp.float32),
      mesh=_mesh,
      scratch_types=[
          pltpu.VMEM((RB, K), jnp.int32),
          pltpu.VMEM((RB, K), jnp.int32),
          pltpu.VMEM((K, 128), jnp.float32),
          pltpu.VMEM((K, 128), jnp.float32),
          pltpu.VMEM_SHARED((ACC_ROWS, 128), jnp.float32),
          pltpu.SemaphoreType.DMA,
          pltpu.SemaphoreType.DMA,
          pltpu.SemaphoreType.DMA,
          pltpu.SemaphoreType.DMA,
      ],
  )
  def prop(y0_hbm, y1_hbm, src2_hbm, dst2_hbm, zeros_hbm,
           out_hbm, srcv, dstv, rows0, rows1, acc, sem0, sem1, sem2, sem3):
    c = lax.axis_index("c")
    s = lax.axis_index("s")
    _tilewise_copy(lambda o, n: zeros_hbm.at[pl.ds(o, n)],
                   lambda o, n: acc.at[pl.ds(o, n)], s)
    plsc.subcore_barrier()

    @pl.when(c == 0)
    def _():
      _edge_pipeline(s * rpt_edges, rpt_edges, y0_hbm, src2_hbm, dst2_hbm,
                     srcv, dstv, (rows0, rows1), acc, (sem0, sem1),
                     (sem2, sem3))

    @pl.when(c == 1)
    def _():
      _edge_pipeline(s * rpt_edges, rpt_edges, y1_hbm, src2_hbm, dst2_hbm,
                     srcv, dstv, (rows0, rows1), acc, (sem0, sem1),
                     (sem2, sem3))

    plsc.subcore_barrier()
    base = pl.multiple_of(c * N, 8)
    _tilewise_copy(lambda o, n: acc.at[pl.ds(o, n)],
                   lambda o, n: out_hbm.at[pl.ds(base + o, n)], s)

  return prop


def _prop_full():
  rpt_edges = CHUNK_ROWS // (NC * NS)

  @functools.partial(
      pl.kernel,
      out_type=jax.ShapeDtypeStruct((NC * N, 128), jnp.float32),
      mesh=_mesh,
      scratch_types=[
          pltpu.VMEM((RB, K), jnp.int32),
          pltpu.VMEM((RB, K), jnp.int32),
          pltpu.VMEM((K, 128), jnp.float32),
          pltpu.VMEM((K, 128), jnp.float32),
          pltpu.VMEM_SHARED((ACC_ROWS, 128), jnp.float32),
          pltpu.SemaphoreType.DMA,
          pltpu.SemaphoreType.DMA,
          pltpu.SemaphoreType.DMA,
          pltpu.SemaphoreType.DMA,
      ],
  )
  def prop(ya_hbm, yb_hbm, src2_hbm, dst2_hbm, zeros_hbm,
           out_hbm, srcv, dstv, rows0, rows1, acc, sem0, sem1, sem2, sem3):
    c = lax.axis_index("c")
    s = lax.axis_index("s")
    _tilewise_copy(lambda o, n: zeros_hbm.at[pl.ds(o, n)],
                   lambda o, n: acc.at[pl.ds(o, n)], s)
    plsc.subcore_barrier()

    @pl.when(c == 0)
    def _():
      _edge_pipeline(s * rpt_edges, rpt_edges, ya_hbm,
                     src2_hbm, dst2_hbm, srcv, dstv, (rows0, rows1), acc,
                     (sem0, sem1), (sem2, sem3))

    @pl.when(c == 1)
    def _():
      _edge_pipeline((NS + s) * rpt_edges, rpt_edges, yb_hbm,
                     src2_hbm, dst2_hbm, srcv, dstv, (rows0, rows1), acc,
                     (sem0, sem1), (sem2, sem3))
    plsc.subcore_barrier()
    base = pl.multiple_of(c * N, 8)
    _tilewise_copy(lambda o, n: acc.at[pl.ds(o, n)],
                   lambda o, n: out_hbm.at[pl.ds(base + o, n)], s)

  return prop


_prop256 = _prop_split()
_prop128 = _prop_full()


@functools.partial(
    pl.kernel,
    out_type=jax.ShapeDtypeStruct((NC * N, DEGW), jnp.float32),
    mesh=_mesh,
    scratch_types=[
        pltpu.VMEM((CHUNK_ROWS // NS, K), jnp.int32),
        pltpu.VMEM((K, DEGW), jnp.float32),
        pltpu.VMEM_SHARED((ACC_ROWS, DEGW), jnp.float32),
        pltpu.SemaphoreType.DMA,
    ],
)
def _deg_kernel(srcc2_hbm, dstp2_hbm, zeros_hbm, ones_hbm,
                out_hbm, idxv, onesv, acc, dsem):
  c = lax.axis_index("c")
  s = lax.axis_index("s")
  rpt_edges = CHUNK_ROWS // NS
  _tilewise_copy(lambda o, n: zeros_hbm.at[pl.ds(o, n)],
                 lambda o, n: acc.at[pl.ds(o, n)], s)
  pltpu.sync_copy(ones_hbm, onesv)
  r0 = pl.multiple_of(s * rpt_edges, 8)

  @pl.when(c == 0)
  def _():
    pltpu.sync_copy(srcc2_hbm.at[pl.ds(r0, rpt_edges)], idxv)

  @pl.when(c == 1)
  def _():
    pltpu.sync_copy(dstp2_hbm.at[pl.ds(r0, rpt_edges)], idxv)

  plsc.subcore_barrier()

  for j in range(4):
    pltpu.async_copy(onesv, acc.at[idxv.at[j]], dsem, add=True)

  def chunk(j, carry):
    pltpu.async_copy(onesv, acc.at[idxv.at[j]], dsem, add=True)
    pltpu.make_async_copy(onesv, acc.at[idxv.at[0]], dsem).wait()
    return carry

  lax.fori_loop(4, rpt_edges, chunk, 0)
  for _ in range(4):
    pltpu.make_async_copy(onesv, acc.at[idxv.at[0]], dsem).wait()
  plsc.subcore_barrier()
  base = pl.multiple_of(c * N, 8)
  _tilewise_copy(lambda o, n: acc.at[pl.ds(o, n)],
                 lambda o, n: out_hbm.at[pl.ds(base + o, n)], s)


def _norm(deg_col):
  return jnp.where(deg_col > 0.0, lax.rsqrt(jnp.maximum(deg_col, 1.0)), 0.0)


def _tc_pre_body(x_ref, deg_ref, y_ref, y2_ref):
  ns = _norm(deg_ref[:, 0:1])
  y = x_ref[...] * ns
  y_ref[...] = y
  y2_ref[...] = y


def _tc_pre(x, deg8):
  return pl.pallas_call(
      _tc_pre_body,
      grid=(GRID,),
      in_specs=[
          pl.BlockSpec((BR, 128), lambda i: (i, 0)),
          pl.BlockSpec((BR, DEGW), lambda i: (i, 0)),
      ],
      out_specs=[pl.BlockSpec((BR, 128), lambda i: (i, 0))] * 2,
      out_shape=[jax.ShapeDtypeStruct((N, 128), jnp.float32)] * 2,
  )(x, deg8)


def _tc_layer_body(mode, fout, split_y, p0_ref, p1_ref, dego_ref, degi_ref,
                   w_ref, b_ref, w4_ref, *out_refs):
  nd = _norm(degi_ref[:, 0:1])
  if mode == "sum":
    a = (p0_ref[...] + p1_ref[...]) * nd
    acc = jnp.dot(a, w_ref[...], preferred_element_type=jnp.float32)
  else:
    acc = jnp.dot(p0_ref[...] * nd, w_ref[:128, :],
                  preferred_element_type=jnp.float32)
    acc += jnp.dot(p1_ref[...] * nd, w_ref[128:, :],
                   preferred_element_type=jnp.float32)
  h = jnp.maximum(acc + b_ref[...], 0.0)
  out_refs[0][...] = h
  ns = _norm(dego_ref[:, 0:1])
  y = h * ns
  if w4_ref is not None:
    y = jnp.dot(y, w4_ref[...], preferred_element_type=jnp.float32)
  if split_y:
    out_refs[1][...] = y[:, :128]
    out_refs[2][...] = y[:, 128:]
  else:
    out_refs[1][...] = y
    if len(out_refs) > 2:
      out_refs[2][...] = y


def _tc_layer(p, deg8, w, b, mode, fout, split_y, w4=None):
  fin = w.shape[0]
  yw = w4.shape[1] if w4 is not None else fout
  in_specs = [
      pl.BlockSpec((BR, 128), lambda i: (i, 0)),
      pl.BlockSpec((BR, 128), lambda i: (i + GRID, 0)),
      pl.BlockSpec((BR, DEGW), lambda i: (i, 0)),
      pl.BlockSpec((BR, DEGW), lambda i: (i + GRID, 0)),
      pl.BlockSpec((fin, fout), lambda i: (0, 0)),
      pl.BlockSpec((1, fout), lambda i: (0, 0)),
  ]
  args = [p, p, deg8, deg8, w, b]
  if w4 is not None:
    in_specs.append(pl.BlockSpec(w4.shape, lambda i: (0, 0)))
    args.append(w4)
  out_specs = [pl.BlockSpec((BR, fout), lambda i: (i, 0))]
  out_shape = [jax.ShapeDtypeStruct((N, fout), jnp.float32)]
  if split_y:
    out_specs += [pl.BlockSpec((BR, 128), lambda i: (i, 0))] * 2
    out_shape += [jax.ShapeDtypeStruct((N, 128), jnp.float32)] * 2
  else:
    ncopy = 2 if w4 is not None else 1
    out_specs += [pl.BlockSpec((BR, yw), lambda i: (i, 0))] * ncopy
    out_shape += [jax.ShapeDtypeStruct((N, yw), jnp.float32)] * ncopy

  body = functools.partial(_tc_layer_body, mode, fout, split_y)
  if w4 is None:
    body2 = lambda a0, a1, a2, a3, a4, a5, *o: body(a0, a1, a2, a3, a4, a5,
                                                    None, *o)
  else:
    body2 = body
  return pl.pallas_call(
      body2,
      grid=(GRID,),
      in_specs=in_specs,
      out_specs=out_specs,
      out_shape=out_shape,
  )(*args)


def _tc_post_body(p0_ref, p1_ref, degi_ref, b_ref, h_ref):
  nd = _norm(degi_ref[:, 0:1])
  h_ref[...] = (p0_ref[...] + p1_ref[...]) * nd + b_ref[...]


def _tc_post(p4, deg8, b4):
  return pl.pallas_call(
      _tc_post_body,
      grid=(GRID,),
      in_specs=[
          pl.BlockSpec((BR, 128), lambda i: (i, 0)),
          pl.BlockSpec((BR, 128), lambda i: (i + GRID, 0)),
          pl.BlockSpec((BR, DEGW), lambda i: (i + GRID, 0)),
          pl.BlockSpec((1, 128), lambda i: (0, 0)),
      ],
      out_specs=pl.BlockSpec((BR, 128), lambda i: (i, 0)),
      out_shape=jax.ShapeDtypeStruct((N, 128), jnp.float32),
  )(p4, p4, deg8, b4)


def kernel(inputs, edge_index, W1, b1, W2, b2, W3, b3, W4, b4):
  src = edge_index[0]
  dst = edge_index[1]
  srcg2 = jnp.concatenate(
      [src, jnp.zeros((EPAD,), jnp.int32)]).reshape(CHUNK_ROWS, K)
  trash_idx = N + (jnp.arange(EPAD, dtype=jnp.int32) % NTRASH)
  srcc2 = jnp.concatenate([src, trash_idx]).reshape(CHUNK_ROWS, K)
  dstp2 = jnp.concatenate([dst, trash_idx]).reshape(CHUNK_ROWS, K)
  zeros128 = jnp.zeros((N, 128), jnp.float32)
  zerosdeg = jnp.zeros((N, DEGW), jnp.float32)
  onesk = jnp.ones((K, DEGW), jnp.float32)

  deg8 = _deg_kernel(srcc2, dstp2, zerosdeg, onesk)

  y1a, y1b = _tc_pre(inputs, deg8[:N])
  p1 = _prop128(y1a, y1b, srcg2, dstp2, zeros128)
  h1, y2lo, y2hi = _tc_layer(p1, deg8, W1, b1.reshape(1, -1), "sum", 256, True)
  p2 = _prop256(y2lo, y2hi, srcg2, dstp2, zeros128)
  h2, y3lo, y3hi = _tc_layer(p2, deg8, W2, b2.reshape(1, -1), "split", 256, True)
  p3 = _prop256(y3lo, y3hi, srcg2, dstp2, zeros128)
  h3, t4a, t4b = _tc_layer(p3, deg8, W3, b3.reshape(1, -1), "split", 256, False,
                           W4)
  p4 = _prop128(t4a, t4b, srcg2, dstp2, zeros128)
  h4 = _tc_post(p4, deg8, b4.reshape(1, -1))
  return (h4, h3, h2, h1)


# speedup vs baseline: 1.0575x; 1.0575x over previous
import functools

import jax
import jax.numpy as jnp
from jax import lax
from jax.experimental import pallas as pl
from jax.experimental.pallas import tpu as pltpu
from jax.experimental.pallas import tpu_sc as plsc

N = 10000
E = 320000
NC = 2
NS = 16
K = 128
CHUNK_ROWS = 2560
EPAD = CHUNK_ROWS * K - E
NTRASH = 128
ACC_ROWS = N + NTRASH
RPT = 624
DEGW = 128

BR = 1000
GRID = N // BR

_mesh = plsc.VectorSubcoreMesh(
    core_axis_name="c", subcore_axis_name="s", num_cores=NC, num_subcores=NS)


def _tilewise_copy(src_at, dst_at, s):
  off = pl.multiple_of(s * RPT, 8)
  pltpu.sync_copy(src_at(off, RPT), dst_at(off, RPT))

  @pl.when(s == NS - 1)
  def _():
    pltpu.sync_copy(src_at(NS * RPT, N - NS * RPT), dst_at(NS * RPT, N - NS * RPT))


RB = 40


def _edge_pipeline(start_row, rows_per_tile, y_hbm, src2_hbm, dst2_hbm,
                   srcv, dstv, rows2, acc, gsems, ssems):
  rows0, rows1 = rows2
  gs0, gs1 = gsems
  ss0, ss1 = ssems
  H = K // 2

  def g_issue(j, buf, sem):
    pltpu.async_copy(y_hbm.at[srcv.at[j, pl.ds(0, H)]],
                     buf.at[pl.ds(0, H)], sem)
    pltpu.async_copy(y_hbm.at[srcv.at[j, pl.ds(H, H)]],
                     buf.at[pl.ds(H, H)], sem)

  def g_wait(buf, sem):
    pltpu.make_async_copy(y_hbm.at[srcv.at[0, pl.ds(0, H)]],
                          buf.at[pl.ds(0, H)], sem).wait()
    pltpu.make_async_copy(y_hbm.at[srcv.at[0, pl.ds(H, H)]],
                          buf.at[pl.ds(H, H)], sem).wait()

  def s_issue(j, buf, sem):
    pltpu.async_copy(buf, acc.at[dstv.at[j]], sem, add=True)

  def s_wait(buf, sem):
    pltpu.make_async_copy(buf, acc.at[dstv.at[0]], sem).wait()

  def blk(b, carry):
    r0 = pl.multiple_of(start_row + b * RB, 8)
    pltpu.sync_copy(src2_hbm.at[pl.ds(r0, RB)], srcv)
    pltpu.sync_copy(dst2_hbm.at[pl.ds(r0, RB)], dstv)
    g_issue(0, rows0, gs0)
    g_issue(1, rows1, gs1)
    g_wait(rows0, gs0)
    s_issue(0, rows0, ss0)
    g_wait(rows1, gs1)
    s_issue(1, rows1, ss1)

    def pair(i, c):
      j0 = i * 2
      s_wait(rows0, ss0)
      g_issue(j0, rows0, gs0)
      s_wait(rows1, ss1)
      g_issue(j0 + 1, rows1, gs1)
      g_wait(rows0, gs0)
      s_issue(j0, rows0, ss0)
      g_wait(rows1, gs1)
      s_issue(j0 + 1, rows1, ss1)
      return c

    lax.fori_loop(1, RB // 2, pair, 0)
    s_wait(rows0, ss0)
    s_wait(rows1, ss1)
    return carry

  lax.fori_loop(0, rows_per_tile // RB, blk, 0)


def _prop_split():
  rpt_edges = CHUNK_ROWS // NS

  @functools.partial(
      pl.kernel,
      out_type=jax.ShapeDtypeStruct((NC * N, 128), jnp.float32),
      mesh=_mesh,
      scratch_types=[
          pltpu.VMEM((RB, K), jnp.int32),
          pltpu.VMEM((RB, K), jnp.int32),
          pltpu.VMEM((K, 128), jnp.float32),
          pltpu.VMEM((K, 128), jnp.float32),
          pltpu.VMEM_SHARED((ACC_ROWS, 128), jnp.float32),
          pltpu.SemaphoreType.DMA,
          pltpu.SemaphoreType.DMA,
          pltpu.SemaphoreType.DMA,
          pltpu.SemaphoreType.DMA,
      ],
  )
  def prop(y0_hbm, y1_hbm, src2_hbm, dst2_hbm, zeros_hbm,
           out_hbm, srcv, dstv, rows0, rows1, acc, sem0, sem1, sem2, sem3):
    c = lax.axis_index("c")
    s = lax.axis_index("s")
    _tilewise_copy(lambda o, n: zeros_hbm.at[pl.ds(o, n)],
                   lambda o, n: acc.at[pl.ds(o, n)], s)
    plsc.subcore_barrier()

    @pl.when(c == 0)
    def _():
      _edge_pipeline(s * rpt_edges, rpt_edges, y0_hbm, src2_hbm, dst2_hbm,
                     srcv, dstv, (rows0, rows1), acc, (sem0, sem1),
                     (sem2, sem3))

    @pl.when(c == 1)
    def _():
      _edge_pipeline(s * rpt_edges, rpt_edges, y1_hbm, src2_hbm, dst2_hbm,
                     srcv, dstv, (rows0, rows1), acc, (sem0, sem1),
                     (sem2, sem3))

    plsc.subcore_barrier()
    base = pl.multiple_of(c * N, 8)
    _tilewise_copy(lambda o, n: acc.at[pl.ds(o, n)],
                   lambda o, n: out_hbm.at[pl.ds(base + o, n)], s)

  return prop


def _prop_full():
  rpt_edges = CHUNK_ROWS // (NC * NS)

  @functools.partial(
      pl.kernel,
      out_type=jax.ShapeDtypeStruct((NC * N, 128), jnp.float32),
      mesh=_mesh,
      scratch_types=[
          pltpu.VMEM((RB, K), jnp.int32),
          pltpu.VMEM((RB, K), jnp.int32),
          pltpu.VMEM((K, 128), jnp.float32),
          pltpu.VMEM((K, 128), jnp.float32),
          pltpu.VMEM_SHARED((ACC_ROWS, 128), jnp.float32),
          pltpu.SemaphoreType.DMA,
          pltpu.SemaphoreType.DMA,
          pltpu.SemaphoreType.DMA,
          pltpu.SemaphoreType.DMA,
      ],
  )
  def prop(y_hbm, src2_hbm, dst2_hbm, zeros_hbm,
           out_hbm, srcv, dstv, rows0, rows1, acc, sem0, sem1, sem2, sem3):
    c = lax.axis_index("c")
    s = lax.axis_index("s")
    _tilewise_copy(lambda o, n: zeros_hbm.at[pl.ds(o, n)],
                   lambda o, n: acc.at[pl.ds(o, n)], s)
    plsc.subcore_barrier()
    _edge_pipeline((c * NS + s) * rpt_edges, rpt_edges, y_hbm,
                   src2_hbm, dst2_hbm, srcv, dstv, (rows0, rows1), acc,
                   (sem0, sem1), (sem2, sem3))
    plsc.subcore_barrier()
    base = pl.multiple_of(c * N, 8)
    _tilewise_copy(lambda o, n: acc.at[pl.ds(o, n)],
                   lambda o, n: out_hbm.at[pl.ds(base + o, n)], s)

  return prop


_prop256 = _prop_split()
_prop128 = _prop_full()


@functools.partial(
    pl.kernel,
    out_type=jax.ShapeDtypeStruct((NC * N, DEGW), jnp.float32),
    mesh=_mesh,
    scratch_types=[
        pltpu.VMEM((CHUNK_ROWS // NS, K), jnp.int32),
        pltpu.VMEM((K, DEGW), jnp.float32),
        pltpu.VMEM_SHARED((ACC_ROWS, DEGW), jnp.float32),
        pltpu.SemaphoreType.DMA,
    ],
)
def _deg_kernel(srcc2_hbm, dstp2_hbm, zeros_hbm, ones_hbm,
                out_hbm, idxv, onesv, acc, dsem):
  c = lax.axis_index("c")
  s = lax.axis_index("s")
  rpt_edges = CHUNK_ROWS // NS
  _tilewise_copy(lambda o, n: zeros_hbm.at[pl.ds(o, n)],
                 lambda o, n: acc.at[pl.ds(o, n)], s)
  pltpu.sync_copy(ones_hbm, onesv)
  r0 = pl.multiple_of(s * rpt_edges, 8)

  @pl.when(c == 0)
  def _():
    pltpu.sync_copy(srcc2_hbm.at[pl.ds(r0, rpt_edges)], idxv)

  @pl.when(c == 1)
  def _():
    pltpu.sync_copy(dstp2_hbm.at[pl.ds(r0, rpt_edges)], idxv)

  plsc.subcore_barrier()

  for j in range(4):
    pltpu.async_copy(onesv, acc.at[idxv.at[j]], dsem, add=True)

  def chunk(j, carry):
    pltpu.async_copy(onesv, acc.at[idxv.at[j]], dsem, add=True)
    pltpu.make_async_copy(onesv, acc.at[idxv.at[0]], dsem).wait()
    return carry

  lax.fori_loop(4, rpt_edges, chunk, 0)
  for _ in range(4):
    pltpu.make_async_copy(onesv, acc.at[idxv.at[0]], dsem).wait()
  plsc.subcore_barrier()
  base = pl.multiple_of(c * N, 8)
  _tilewise_copy(lambda o, n: acc.at[pl.ds(o, n)],
                 lambda o, n: out_hbm.at[pl.ds(base + o, n)], s)


def _norm(deg_col):
  return jnp.where(deg_col > 0.0, lax.rsqrt(jnp.maximum(deg_col, 1.0)), 0.0)


def _tc_pre_body(x_ref, deg_ref, y_ref):
  ns = _norm(deg_ref[:, 0:1])
  y_ref[...] = x_ref[...] * ns


def _tc_pre(x, deg8):
  return pl.pallas_call(
      _tc_pre_body,
      grid=(GRID,),
      in_specs=[
          pl.BlockSpec((BR, 128), lambda i: (i, 0)),
          pl.BlockSpec((BR, DEGW), lambda i: (i, 0)),
      ],
      out_specs=pl.BlockSpec((BR, 128), lambda i: (i, 0)),
      out_shape=jax.ShapeDtypeStruct((N, 128), jnp.float32),
  )(x, deg8)


def _tc_layer_body(mode, fout, split_y, p0_ref, p1_ref, dego_ref, degi_ref,
                   w_ref, b_ref, w4_ref, *out_refs):
  nd = _norm(degi_ref[:, 0:1])
  if mode == "sum":
    a = (p0_ref[...] + p1_ref[...]) * nd
    acc = jnp.dot(a, w_ref[...], preferred_element_type=jnp.float32)
  else:
    acc = jnp.dot(p0_ref[...] * nd, w_ref[:128, :],
                  preferred_element_type=jnp.float32)
    acc += jnp.dot(p1_ref[...] * nd, w_ref[128:, :],
                   preferred_element_type=jnp.float32)
  h = jnp.maximum(acc + b_ref[...], 0.0)
  out_refs[0][...] = h
  ns = _norm(dego_ref[:, 0:1])
  y = h * ns
  if w4_ref is not None:
    y = jnp.dot(y, w4_ref[...], preferred_element_type=jnp.float32)
  if split_y:
    out_refs[1][...] = y[:, :128]
    out_refs[2][...] = y[:, 128:]
  else:
    out_refs[1][...] = y


def _tc_layer(p, deg8, w, b, mode, fout, split_y, w4=None):
  fin = w.shape[0]
  yw = w4.shape[1] if w4 is not None else fout
  in_specs = [
      pl.BlockSpec((BR, 128), lambda i: (i, 0)),
      pl.BlockSpec((BR, 128), lambda i: (i + GRID, 0)),
      pl.BlockSpec((BR, DEGW), lambda i: (i, 0)),
      pl.BlockSpec((BR, DEGW), lambda i: (i + GRID, 0)),
      pl.BlockSpec((fin, fout), lambda i: (0, 0)),
      pl.BlockSpec((1, fout), lambda i: (0, 0)),
  ]
  args = [p, p, deg8, deg8, w, b]
  if w4 is not None:
    in_specs.append(pl.BlockSpec(w4.shape, lambda i: (0, 0)))
    args.append(w4)
  out_specs = [pl.BlockSpec((BR, fout), lambda i: (i, 0))]
  out_shape = [jax.ShapeDtypeStruct((N, fout), jnp.float32)]
  if split_y:
    out_specs += [pl.BlockSpec((BR, 128), lambda i: (i, 0))] * 2
    out_shape += [jax.ShapeDtypeStruct((N, 128), jnp.float32)] * 2
  else:
    out_specs.append(pl.BlockSpec((BR, yw), lambda i: (i, 0)))
    out_shape.append(jax.ShapeDtypeStruct((N, yw), jnp.float32))

  body = functools.partial(_tc_layer_body, mode, fout, split_y)
  if w4 is None:
    body2 = lambda a0, a1, a2, a3, a4, a5, *o: body(a0, a1, a2, a3, a4, a5,
                                                    None, *o)
  else:
    body2 = body
  return pl.pallas_call(
      body2,
      grid=(GRID,),
      in_specs=in_specs,
      out_specs=out_specs,
      out_shape=out_shape,
  )(*args)


def _tc_post_body(p0_ref, p1_ref, degi_ref, b_ref, h_ref):
  nd = _norm(degi_ref[:, 0:1])
  h_ref[...] = (p0_ref[...] + p1_ref[...]) * nd + b_ref[...]


def _tc_post(p4, deg8, b4):
  return pl.pallas_call(
      _tc_post_body,
      grid=(GRID,),
      in_specs=[
          pl.BlockSpec((BR, 128), lambda i: (i, 0)),
          pl.BlockSpec((BR, 128), lambda i: (i + GRID, 0)),
          pl.BlockSpec((BR, DEGW), lambda i: (i + GRID, 0)),
          pl.BlockSpec((1, 128), lambda i: (0, 0)),
      ],
      out_specs=pl.BlockSpec((BR, 128), lambda i: (i, 0)),
      out_shape=jax.ShapeDtypeStruct((N, 128), jnp.float32),
  )(p4, p4, deg8, b4)


def kernel(inputs, edge_index, W1, b1, W2, b2, W3, b3, W4, b4):
  src = edge_index[0]
  dst = edge_index[1]
  srcg2 = jnp.concatenate(
      [src, jnp.zeros((EPAD,), jnp.int32)]).reshape(CHUNK_ROWS, K)
  trash_idx = N + (jnp.arange(EPAD, dtype=jnp.int32) % NTRASH)
  srcc2 = jnp.concatenate([src, trash_idx]).reshape(CHUNK_ROWS, K)
  dstp2 = jnp.concatenate([dst, trash_idx]).reshape(CHUNK_ROWS, K)
  zeros128 = jnp.zeros((N, 128), jnp.float32)
  zerosdeg = jnp.zeros((N, DEGW), jnp.float32)
  onesk = jnp.ones((K, DEGW), jnp.float32)

  deg8 = _deg_kernel(srcc2, dstp2, zerosdeg, onesk)

  y1 = _tc_pre(inputs, deg8[:N])
  p1 = _prop128(y1, srcg2, dstp2, zeros128)
  h1, y2lo, y2hi = _tc_layer(p1, deg8, W1, b1.reshape(1, -1), "sum", 256, True)
  p2 = _prop256(y2lo, y2hi, srcg2, dstp2, zeros128)
  h2, y3lo, y3hi = _tc_layer(p2, deg8, W2, b2.reshape(1, -1), "split", 256, True)
  p3 = _prop256(y3lo, y3hi, srcg2, dstp2, zeros128)
  h3, t4 = _tc_layer(p3, deg8, W3, b3.reshape(1, -1), "split", 256, False, W4)
  p4 = _prop128(t4, srcg2, dstp2, zeros128)
  h4 = _tc_post(p4, deg8, b4.reshape(1, -1))
  return (h4, h3, h2, h1)


# speedup vs baseline: 2.3580x; 2.2298x over previous
import functools

import jax
import jax.numpy as jnp
from jax import lax
from jax.experimental import pallas as pl
from jax.experimental.pallas import tpu as pltpu
from jax.experimental.pallas import tpu_sc as plsc

N = 10000
E = 320000
NC = 2
NS = 16
K = 128
CHUNK_ROWS = 2560
EPAD = CHUNK_ROWS * K - E
NTRASH = 128
ACC_ROWS = N + NTRASH
RPT = 624
DEGW = 128

BR = 1000
GRID = N // BR

_mesh = plsc.VectorSubcoreMesh(
    core_axis_name="c", subcore_axis_name="s", num_cores=NC, num_subcores=NS)


def _tilewise_copy(src_at, dst_at, s):
  off = pl.multiple_of(s * RPT, 8)
  pltpu.sync_copy(src_at(off, RPT), dst_at(off, RPT))

  @pl.when(s == NS - 1)
  def _():
    pltpu.sync_copy(src_at(NS * RPT, N - NS * RPT), dst_at(NS * RPT, N - NS * RPT))


RB = 40


def _edge_pipeline(start_row, rows_per_tile, y_hbm, src2_hbm, dst2_hbm,
                   srcv, dstv, rows2, acc, gsems, ssems):
  rows0, rows1 = rows2
  gs0, gs1 = gsems
  ss0, ss1 = ssems
  H = K // 2

  def g_issue(j, buf, sem):
    pltpu.async_copy(y_hbm.at[srcv.at[j, pl.ds(0, H)]],
                     buf.at[pl.ds(0, H)], sem)
    pltpu.async_copy(y_hbm.at[srcv.at[j, pl.ds(H, H)]],
                     buf.at[pl.ds(H, H)], sem)

  def g_wait(buf, sem):
    pltpu.make_async_copy(y_hbm.at[srcv.at[0, pl.ds(0, H)]],
                          buf.at[pl.ds(0, H)], sem).wait()
    pltpu.make_async_copy(y_hbm.at[srcv.at[0, pl.ds(H, H)]],
                          buf.at[pl.ds(H, H)], sem).wait()

  def s_issue(j, buf, sem):
    pltpu.async_copy(buf, acc.at[dstv.at[j]], sem, add=True)

  def s_wait(buf, sem):
    pltpu.make_async_copy(buf, acc.at[dstv.at[0]], sem).wait()

  def blk(b, carry):
    r0 = pl.multiple_of(start_row + b * RB, 8)
    pltpu.sync_copy(src2_hbm.at[pl.ds(r0, RB)], srcv)
    pltpu.sync_copy(dst2_hbm.at[pl.ds(r0, RB)], dstv)
    g_issue(0, rows0, gs0)
    g_issue(1, rows1, gs1)
    g_wait(rows0, gs0)
    s_issue(0, rows0, ss0)
    g_wait(rows1, gs1)
    s_issue(1, rows1, ss1)

    def pair(i, c):
      j0 = i * 2
      s_wait(rows0, ss0)
      g_issue(j0, rows0, gs0)
      s_wait(rows1, ss1)
      g_issue(j0 + 1, rows1, gs1)
      g_wait(rows0, gs0)
      s_issue(j0, rows0, ss0)
      g_wait(rows1, gs1)
      s_issue(j0 + 1, rows1, ss1)
      return c

    lax.fori_loop(1, RB // 2, pair, 0)
    s_wait(rows0, ss0)
    s_wait(rows1, ss1)
    return carry

  lax.fori_loop(0, rows_per_tile // RB, blk, 0)


def _prop_split():
  rpt_edges = CHUNK_ROWS // NS

  @functools.partial(
      pl.kernel,
      out_type=jax.ShapeDtypeStruct((NC * N, 128), jnp.float32),
      mesh=_mesh,
      scratch_types=[
          pltpu.VMEM((RB, K), jnp.int32),
          pltpu.VMEM((RB, K), jnp.int32),
          pltpu.VMEM((K, 128), jnp.float32),
          pltpu.VMEM((K, 128), jnp.float32),
          pltpu.VMEM_SHARED((ACC_ROWS, 128), jnp.float32),
          pltpu.SemaphoreType.DMA,
          pltpu.SemaphoreType.DMA,
          pltpu.SemaphoreType.DMA,
          pltpu.SemaphoreType.DMA,
      ],
  )
  def prop(y0_hbm, y1_hbm, src2_hbm, dst2_hbm, zeros_hbm,
           out_hbm, srcv, dstv, rows0, rows1, acc, sem0, sem1, sem2, sem3):
    c = lax.axis_index("c")
    s = lax.axis_index("s")
    _tilewise_copy(lambda o, n: zeros_hbm.at[pl.ds(o, n)],
                   lambda o, n: acc.at[pl.ds(o, n)], s)
    plsc.subcore_barrier()

    @pl.when(c == 0)
    def _():
      _edge_pipeline(s * rpt_edges, rpt_edges, y0_hbm, src2_hbm, dst2_hbm,
                     srcv, dstv, (rows0, rows1), acc, (sem0, sem1),
                     (sem2, sem3))

    @pl.when(c == 1)
    def _():
      _edge_pipeline(s * rpt_edges, rpt_edges, y1_hbm, src2_hbm, dst2_hbm,
                     srcv, dstv, (rows0, rows1), acc, (sem0, sem1),
                     (sem2, sem3))

    plsc.subcore_barrier()
    base = pl.multiple_of(c * N, 8)
    _tilewise_copy(lambda o, n: acc.at[pl.ds(o, n)],
                   lambda o, n: out_hbm.at[pl.ds(base + o, n)], s)

  return prop


def _prop_full():
  rpt_edges = CHUNK_ROWS // (NC * NS)

  @functools.partial(
      pl.kernel,
      out_type=jax.ShapeDtypeStruct((NC * N, 128), jnp.float32),
      mesh=_mesh,
      scratch_types=[
          pltpu.VMEM((RB, K), jnp.int32),
          pltpu.VMEM((RB, K), jnp.int32),
          pltpu.VMEM((K, 128), jnp.float32),
          pltpu.VMEM((K, 128), jnp.float32),
          pltpu.VMEM_SHARED((ACC_ROWS, 128), jnp.float32),
          pltpu.SemaphoreType.DMA,
          pltpu.SemaphoreType.DMA,
          pltpu.SemaphoreType.DMA,
          pltpu.SemaphoreType.DMA,
      ],
  )
  def prop(y_hbm, src2_hbm, dst2_hbm, zeros_hbm,
           out_hbm, srcv, dstv, rows0, rows1, acc, sem0, sem1, sem2, sem3):
    c = lax.axis_index("c")
    s = lax.axis_index("s")
    _tilewise_copy(lambda o, n: zeros_hbm.at[pl.ds(o, n)],
                   lambda o, n: acc.at[pl.ds(o, n)], s)
    plsc.subcore_barrier()
    _edge_pipeline((c * NS + s) * rpt_edges, rpt_edges, y_hbm,
                   src2_hbm, dst2_hbm, srcv, dstv, (rows0, rows1), acc,
                   (sem0, sem1), (sem2, sem3))
    plsc.subcore_barrier()
    base = pl.multiple_of(c * N, 8)
    _tilewise_copy(lambda o, n: acc.at[pl.ds(o, n)],
                   lambda o, n: out_hbm.at[pl.ds(base + o, n)], s)

  return prop


_prop256 = _prop_split()
_prop128 = _prop_full()


@functools.partial(
    pl.kernel,
    out_type=jax.ShapeDtypeStruct((NC * N, DEGW), jnp.float32),
    mesh=_mesh,
    scratch_types=[
        pltpu.VMEM((CHUNK_ROWS // NS, K), jnp.int32),
        pltpu.VMEM((K, DEGW), jnp.float32),
        pltpu.VMEM_SHARED((ACC_ROWS, DEGW), jnp.float32),
        pltpu.SemaphoreType.DMA,
    ],
)
def _deg_kernel(srcc2_hbm, dstp2_hbm, zeros_hbm, ones_hbm,
                out_hbm, idxv, onesv, acc, dsem):
  c = lax.axis_index("c")
  s = lax.axis_index("s")
  rpt_edges = CHUNK_ROWS // NS
  _tilewise_copy(lambda o, n: zeros_hbm.at[pl.ds(o, n)],
                 lambda o, n: acc.at[pl.ds(o, n)], s)
  pltpu.sync_copy(ones_hbm, onesv)
  r0 = pl.multiple_of(s * rpt_edges, 8)

  @pl.when(c == 0)
  def _():
    pltpu.sync_copy(srcc2_hbm.at[pl.ds(r0, rpt_edges)], idxv)

  @pl.when(c == 1)
  def _():
    pltpu.sync_copy(dstp2_hbm.at[pl.ds(r0, rpt_edges)], idxv)

  plsc.subcore_barrier()

  for j in range(4):
    pltpu.async_copy(onesv, acc.at[idxv.at[j]], dsem, add=True)

  def chunk(j, carry):
    pltpu.async_copy(onesv, acc.at[idxv.at[j]], dsem, add=True)
    pltpu.make_async_copy(onesv, acc.at[idxv.at[0]], dsem).wait()
    return carry

  lax.fori_loop(4, rpt_edges, chunk, 0)
  for _ in range(4):
    pltpu.make_async_copy(onesv, acc.at[idxv.at[0]], dsem).wait()
  plsc.subcore_barrier()
  base = pl.multiple_of(c * N, 8)
  _tilewise_copy(lambda o, n: acc.at[pl.ds(o, n)],
                 lambda o, n: out_hbm.at[pl.ds(base + o, n)], s)


def _norm(deg_col):
  return jnp.where(deg_col > 0.0, lax.rsqrt(jnp.maximum(deg_col, 1.0)), 0.0)


def _tc_pre_body(x_ref, deg_ref, y_ref):
  ns = _norm(deg_ref[:, 0:1])
  y_ref[...] = x_ref[...] * ns


def _tc_pre(x, deg8):
  return pl.pallas_call(
      _tc_pre_body,
      grid=(GRID,),
      in_specs=[
          pl.BlockSpec((BR, 128), lambda i: (i, 0)),
          pl.BlockSpec((BR, DEGW), lambda i: (i, 0)),
      ],
      out_specs=pl.BlockSpec((BR, 128), lambda i: (i, 0)),
      out_shape=jax.ShapeDtypeStruct((N, 128), jnp.float32),
  )(x, deg8)


def _tc_layer_body(mode, fout, split_y, p0_ref, p1_ref, dego_ref, degi_ref,
                   w_ref, b_ref, w4_ref, *out_refs):
  nd = _norm(degi_ref[:, 0:1])
  if mode == "sum":
    a = (p0_ref[...] + p1_ref[...]) * nd
    acc = jnp.dot(a, w_ref[...], preferred_element_type=jnp.float32)
  else:
    acc = jnp.dot(p0_ref[...] * nd, w_ref[:128, :],
                  preferred_element_type=jnp.float32)
    acc += jnp.dot(p1_ref[...] * nd, w_ref[128:, :],
                   preferred_element_type=jnp.float32)
  h = jnp.maximum(acc + b_ref[...], 0.0)
  out_refs[0][...] = h
  ns = _norm(dego_ref[:, 0:1])
  y = h * ns
  if w4_ref is not None:
    y = jnp.dot(y, w4_ref[...], preferred_element_type=jnp.float32)
  if split_y:
    out_refs[1][...] = y[:, :128]
    out_refs[2][...] = y[:, 128:]
  else:
    out_refs[1][...] = y


def _tc_layer(p, deg8, w, b, mode, fout, split_y, w4=None):
  fin = w.shape[0]
  yw = w4.shape[1] if w4 is not None else fout
  in_specs = [
      pl.BlockSpec((BR, 128), lambda i: (i, 0)),
      pl.BlockSpec((BR, 128), lambda i: (i + GRID, 0)),
      pl.BlockSpec((BR, DEGW), lambda i: (i, 0)),
      pl.BlockSpec((BR, DEGW), lambda i: (i + GRID, 0)),
      pl.BlockSpec((fin, fout), lambda i: (0, 0)),
      pl.BlockSpec((1, fout), lambda i: (0, 0)),
  ]
  args = [p, p, deg8, deg8, w, b]
  if w4 is not None:
    in_specs.append(pl.BlockSpec(w4.shape, lambda i: (0, 0)))
    args.append(w4)
  out_specs = [pl.BlockSpec((BR, fout), lambda i: (i, 0))]
  out_shape = [jax.ShapeDtypeStruct((N, fout), jnp.float32)]
  if split_y:
    out_specs += [pl.BlockSpec((BR, 128), lambda i: (i, 0))] * 2
    out_shape += [jax.ShapeDtypeStruct((N, 128), jnp.float32)] * 2
  else:
    out_specs.append(pl.BlockSpec((BR, yw), lambda i: (i, 0)))
    out_shape.append(jax.ShapeDtypeStruct((N, yw), jnp.float32))

  body = functools.partial(_tc_layer_body, mode, fout, split_y)
  if w4 is None:
    body2 = lambda a0, a1, a2, a3, a4, a5, *o: body(a0, a1, a2, a3, a4, a5,
                                                    None, *o)
  else:
    body2 = body
  return pl.pallas_call(
      body2,
      grid=(GRID,),
      in_specs=in_specs,
      out_specs=out_specs,
      out_shape=out_shape,
  )(*args)


def _tc_post_body(p0_ref, p1_ref, degi_ref, b_ref, h_ref):
  nd = _norm(degi_ref[:, 0:1])
  h_ref[...] = (p0_ref[...] + p1_ref[...]) * nd + b_ref[...]


def _tc_post(p4, deg8, b4):
  return pl.pallas_call(
      _tc_post_body,
      grid=(GRID,),
      in_specs=[
          pl.BlockSpec((BR, 128), lambda i: (i, 0)),
          pl.BlockSpec((BR, 128), lambda i: (i + GRID, 0)),
          pl.BlockSpec((BR, DEGW), lambda i: (i + GRID, 0)),
          pl.BlockSpec((1, 128), lambda i: (0, 0)),
      ],
      out_specs=pl.BlockSpec((BR, 128), lambda i: (i, 0)),
      out_shape=jax.ShapeDtypeStruct((N, 128), jnp.float32),
  )(p4, p4, deg8, b4)


def kernel(inputs, edge_index, W1, b1, W2, b2, W3, b3, W4, b4):
  src = edge_index[0]
  dst = edge_index[1]
  srcg2 = jnp.concatenate(
      [src, jnp.arange(EPAD, dtype=jnp.int32) % N]).reshape(CHUNK_ROWS, K)
  trash_idx = N + (jnp.arange(EPAD, dtype=jnp.int32) % NTRASH)
  srcc2 = jnp.concatenate([src, trash_idx]).reshape(CHUNK_ROWS, K)
  dstp2 = jnp.concatenate([dst, trash_idx]).reshape(CHUNK_ROWS, K)
  zeros128 = jnp.zeros((N, 128), jnp.float32)
  zerosdeg = jnp.zeros((N, DEGW), jnp.float32)
  onesk = jnp.ones((K, DEGW), jnp.float32)

  deg8 = _deg_kernel(srcc2, dstp2, zerosdeg, onesk)

  y1 = _tc_pre(inputs, deg8[:N])
  p1 = _prop128(y1, srcg2, dstp2, zeros128)
  h1, y2lo, y2hi = _tc_layer(p1, deg8, W1, b1.reshape(1, -1), "sum", 256, True)
  p2 = _prop256(y2lo, y2hi, srcg2, dstp2, zeros128)
  h2, y3lo, y3hi = _tc_layer(p2, deg8, W2, b2.reshape(1, -1), "split", 256, True)
  p3 = _prop256(y3lo, y3hi, srcg2, dstp2, zeros128)
  h3, t4 = _tc_layer(p3, deg8, W3, b3.reshape(1, -1), "split", 256, False, W4)
  p4 = _prop128(t4, srcg2, dstp2, zeros128)
  h4 = _tc_post(p4, deg8, b4.reshape(1, -1))
  return (h4, h3, h2, h1)
